# initial kernel scaffold (unmeasured)
import jax
import jax.numpy as jnp
from jax import lax
from jax.experimental import pallas as pl
from jax.experimental.pallas import tpu as pltpu

N_DEV = 32
B = 2
SQ = 256
SKV = 256
HQ_LOC = 4
DH = 64
DM = 512
HD = HQ_LOC * DH


def kernel(x, Wq, K_ext, V_ext, Wo):
    def body(x_ref, wq_ref, k_hbm, v_hbm, wo_ref, out_ref,
             comm_wq, comm_wo, kbuf, vbuf,
             wq_send_sem, wq_recv_sem, wo_send_sem, wo_recv_sem,
             ksem, vsem):
        my = lax.axis_index("i")
        left = jnp.mod(my - 1, N_DEV)
        right = jnp.mod(my + 1, N_DEV)

        bsem = pltpu.get_barrier_semaphore()
        pl.semaphore_signal(bsem, inc=1, device_id=(left,),
                            device_id_type=pl.DeviceIdType.MESH)
        pl.semaphore_signal(bsem, inc=1, device_id=(right,),
                            device_id_type=pl.DeviceIdType.MESH)
        pl.semaphore_wait(bsem, 2)

        comm_wq[0] = wq_ref[...]
        comm_wo[0] = wo_ref[...]
        out_ref[...] = jnp.zeros((B, SQ, DM), jnp.float32)

        r = lax.broadcasted_iota(jnp.int32, (SQ, SKV), 0)
        c = lax.broadcasted_iota(jnp.int32, (SQ, SKV), 1)
        qb = my * (SQ // 64) + r // 64
        kb_blk = c // 64
        mask = (qb == kb_blk) | (kb_blk == 0) | (jnp.mod(qb + kb_blk, 3) == 0)

        x2d = x_ref[...].reshape(B * SQ, DM)

        def step(h, _):
            send_slot = jnp.mod(h, 2)
            recv_slot = jnp.mod(h + 1, 2)
            blk = jnp.mod(my - h, N_DEV)

            rdma_wq = pltpu.make_async_remote_copy(
                src_ref=comm_wq.at[send_slot],
                dst_ref=comm_wq.at[recv_slot],
                send_sem=wq_send_sem.at[send_slot],
                recv_sem=wq_recv_sem.at[recv_slot],
                device_id=(right,),
                device_id_type=pl.DeviceIdType.MESH,
            )
            rdma_wo = pltpu.make_async_remote_copy(
                src_ref=comm_wo.at[send_slot],
                dst_ref=comm_wo.at[recv_slot],
                send_sem=wo_send_sem.at[send_slot],
                recv_sem=wo_recv_sem.at[recv_slot],
                device_id=(right,),
                device_id_type=pl.DeviceIdType.MESH,
            )

            @pl.when(h < N_DEV - 1)
            def _():
                rdma_wq.start()
                rdma_wo.start()

            kcp = pltpu.make_async_copy(
                k_hbm.at[:, :, pl.ds(blk * HQ_LOC, HQ_LOC), :], kbuf, ksem)
            vcp = pltpu.make_async_copy(
                v_hbm.at[:, :, pl.ds(blk * HQ_LOC, HQ_LOC), :], vbuf, vsem)
            kcp.start()
            vcp.start()
            kcp.wait()
            vcp.wait()

            wq_k = comm_wq[send_slot]
            wo_k = comm_wo[send_slot]
            q = jnp.dot(x2d, wq_k, preferred_element_type=jnp.float32)
            q4 = q.reshape(B, SQ, HQ_LOC, DH)
            kv_k = kbuf[...]
            vv_k = vbuf[...]

            ctx_bs = []
            for b in range(B):
                ctx_h = []
                for hh in range(HQ_LOC):
                    q_bh = q4[b, :, hh, :]
                    k_bh = kv_k[b, :, hh, :]
                    s = lax.dot_general(
                        q_bh, k_bh, (((1,), (1,)), ((), ())),
                        preferred_element_type=jnp.float32) * 0.125
                    s = jnp.where(mask, s, -1e9)
                    mx = jnp.max(s, axis=1, keepdims=True)
                    w = jnp.exp(s - mx)
                    w = w / jnp.sum(w, axis=1, keepdims=True)
                    ctx_h.append(jnp.dot(w, vv_k[b, :, hh, :],
                                         preferred_element_type=jnp.float32))
                ctx_bs.append(jnp.concatenate(ctx_h, axis=1))
            ctx2d = jnp.stack(ctx_bs, axis=0).reshape(B * SQ, HD)
            partial = jnp.dot(ctx2d, wo_k, preferred_element_type=jnp.float32)
            out_ref[...] = out_ref[...] + partial.reshape(B, SQ, DM)

            @pl.when(h < N_DEV - 1)
            def _():
                rdma_wq.wait()
                rdma_wo.wait()

            return 0

        lax.fori_loop(0, N_DEV, step, 0)

    return pl.pallas_call(
        body,
        out_shape=jax.ShapeDtypeStruct((B, SQ, DM), jnp.float32),
        in_specs=[
            pl.BlockSpec(memory_space=pltpu.VMEM),
            pl.BlockSpec(memory_space=pltpu.VMEM),
            pl.BlockSpec(memory_space=pltpu.ANY),
            pl.BlockSpec(memory_space=pltpu.ANY),
            pl.BlockSpec(memory_space=pltpu.VMEM),
        ],
        out_specs=pl.BlockSpec(memory_space=pltpu.VMEM),
        scratch_shapes=[
            pltpu.VMEM((2, DM, HD), jnp.float32),
            pltpu.VMEM((2, HD, DM), jnp.float32),
            pltpu.VMEM((B, SKV, HQ_LOC, DH), jnp.float32),
            pltpu.VMEM((B, SKV, HQ_LOC, DH), jnp.float32),
            pltpu.SemaphoreType.DMA((2,)),
            pltpu.SemaphoreType.DMA((2,)),
            pltpu.SemaphoreType.DMA((2,)),
            pltpu.SemaphoreType.DMA((2,)),
            pltpu.SemaphoreType.DMA,
            pltpu.SemaphoreType.DMA,
        ],
        compiler_params=pltpu.CompilerParams(collective_id=0),
    )(x, Wq, K_ext, V_ext, Wo)


# baseline (device time: 459455 ns/iter reference)
import jax
import jax.numpy as jnp
from jax import lax
from jax.experimental import pallas as pl
from jax.experimental.pallas import tpu as pltpu

N_DEV = 32
B = 2
SQ = 256
SKV = 256
HQ_LOC = 4
DH = 64
DM = 512
HD = HQ_LOC * DH


def kernel(x, Wq, K_ext, V_ext, Wo):
    def body(x_ref, wq_ref, k_hbm, v_hbm, wo_ref, out_ref,
             comm_wq, comm_wo, kbuf, vbuf,
             wq_send_sem, wq_recv_sem, wo_send_sem, wo_recv_sem,
             ksem, vsem):
        my = lax.axis_index("i")
        left = jnp.mod(my - 1, N_DEV)
        right = jnp.mod(my + 1, N_DEV)

        bsem = pltpu.get_barrier_semaphore()
        pl.semaphore_signal(bsem, inc=1, device_id=(left,),
                            device_id_type=pl.DeviceIdType.MESH)
        pl.semaphore_signal(bsem, inc=1, device_id=(right,),
                            device_id_type=pl.DeviceIdType.MESH)
        pl.semaphore_wait(bsem, 2)

        comm_wq[0] = wq_ref[...]
        comm_wo[0] = wo_ref[...]
        out_ref[...] = jnp.zeros((B, SQ, DM), jnp.float32)

        r = lax.broadcasted_iota(jnp.int32, (SQ, SKV), 0)
        c = lax.broadcasted_iota(jnp.int32, (SQ, SKV), 1)
        qb = my * (SQ // 64) + r // 64
        kb_blk = c // 64
        mask = (qb == kb_blk) | (kb_blk == 0) | (jnp.mod(qb + kb_blk, 3) == 0)

        x2d = x_ref[...].reshape(B * SQ, DM)

        def step(h, _):
            send_slot = jnp.mod(h, 2)
            recv_slot = jnp.mod(h + 1, 2)
            blk = jnp.mod(my - h, N_DEV)

            rdma_wq = pltpu.make_async_remote_copy(
                src_ref=comm_wq.at[send_slot],
                dst_ref=comm_wq.at[recv_slot],
                send_sem=wq_send_sem.at[send_slot],
                recv_sem=wq_recv_sem.at[recv_slot],
                device_id=(right,),
                device_id_type=pl.DeviceIdType.MESH,
            )
            rdma_wo = pltpu.make_async_remote_copy(
                src_ref=comm_wo.at[send_slot],
                dst_ref=comm_wo.at[recv_slot],
                send_sem=wo_send_sem.at[send_slot],
                recv_sem=wo_recv_sem.at[recv_slot],
                device_id=(right,),
                device_id_type=pl.DeviceIdType.MESH,
            )

            @pl.when(h < N_DEV - 1)
            def _():
                rdma_wq.start()
                rdma_wo.start()

            kcp = pltpu.make_async_copy(
                k_hbm.at[:, :, pl.ds(blk * HQ_LOC, HQ_LOC), :], kbuf, ksem)
            vcp = pltpu.make_async_copy(
                v_hbm.at[:, :, pl.ds(blk * HQ_LOC, HQ_LOC), :], vbuf, vsem)
            kcp.start()
            vcp.start()
            kcp.wait()
            vcp.wait()

            wq_k = comm_wq[send_slot]
            wo_k = comm_wo[send_slot]
            q = jnp.dot(x2d, wq_k, preferred_element_type=jnp.float32)
            q4 = q.reshape(B, SQ, HQ_LOC, DH)
            kv_k = kbuf[...]
            vv_k = vbuf[...]

            ctx_bs = []
            for b in range(B):
                ctx_h = []
                for hh in range(HQ_LOC):
                    q_bh = q4[b, :, hh, :]
                    k_bh = kv_k[b, :, hh, :]
                    s = lax.dot_general(
                        q_bh, k_bh, (((1,), (1,)), ((), ())),
                        preferred_element_type=jnp.float32) * 0.125
                    s = jnp.where(mask, s, -1e9)
                    mx = jnp.max(s, axis=1, keepdims=True)
                    w = jnp.exp(s - mx)
                    w = w / jnp.sum(w, axis=1, keepdims=True)
                    ctx_h.append(jnp.dot(w, vv_k[b, :, hh, :],
                                         preferred_element_type=jnp.float32))
                ctx_bs.append(jnp.concatenate(ctx_h, axis=1))
            ctx2d = jnp.stack(ctx_bs, axis=0).reshape(B * SQ, HD)
            partial = jnp.dot(ctx2d, wo_k, preferred_element_type=jnp.float32)
            out_ref[...] = out_ref[...] + partial.reshape(B, SQ, DM)

            @pl.when(h < N_DEV - 1)
            def _():
                rdma_wq.wait()
                rdma_wo.wait()

            return 0

        lax.fori_loop(0, N_DEV, step, 0)

    return pl.pallas_call(
        body,
        out_shape=jax.ShapeDtypeStruct((B, SQ, DM), jnp.float32),
        in_specs=[
            pl.BlockSpec(memory_space=pltpu.VMEM),
            pl.BlockSpec(memory_space=pltpu.VMEM),
            pl.BlockSpec(memory_space=pltpu.MemorySpace.HBM),
            pl.BlockSpec(memory_space=pltpu.MemorySpace.HBM),
            pl.BlockSpec(memory_space=pltpu.VMEM),
        ],
        out_specs=pl.BlockSpec(memory_space=pltpu.VMEM),
        scratch_shapes=[
            pltpu.VMEM((2, DM, HD), jnp.float32),
            pltpu.VMEM((2, HD, DM), jnp.float32),
            pltpu.VMEM((B, SKV, HQ_LOC, DH), jnp.float32),
            pltpu.VMEM((B, SKV, HQ_LOC, DH), jnp.float32),
            pltpu.SemaphoreType.DMA((2,)),
            pltpu.SemaphoreType.DMA((2,)),
            pltpu.SemaphoreType.DMA((2,)),
            pltpu.SemaphoreType.DMA((2,)),
            pltpu.SemaphoreType.DMA,
            pltpu.SemaphoreType.DMA,
        ],
        compiler_params=pltpu.CompilerParams(collective_id=0),
    )(x, Wq, K_ext, V_ext, Wo)


# device time: 283636 ns/iter; 1.6199x vs baseline; 1.6199x over previous
import jax
import jax.numpy as jnp
from jax import lax
from jax.experimental import pallas as pl
from jax.experimental.pallas import tpu as pltpu

N_DEV = 32
B = 2
SQ = 256
SKV = 256
HQ_LOC = 4
DH = 64
DM = 512
HD = HQ_LOC * DH


def kernel(x, Wq, K_ext, V_ext, Wo):
    def body(x_ref, wq_ref, k_hbm, v_hbm, wo_ref, out_ref,
             comm_wq, comm_wo, kbuf, vbuf,
             wq_send_sem, wq_recv_sem, wo_send_sem, wo_recv_sem,
             ksem, vsem):
        my = lax.axis_index("i")
        left = jnp.mod(my - 1, N_DEV)
        right = jnp.mod(my + 1, N_DEV)

        bsem = pltpu.get_barrier_semaphore()
        pl.semaphore_signal(bsem, inc=1, device_id=(left,),
                            device_id_type=pl.DeviceIdType.MESH)
        pl.semaphore_signal(bsem, inc=1, device_id=(right,),
                            device_id_type=pl.DeviceIdType.MESH)
        pl.semaphore_wait(bsem, 2)

        comm_wq[0] = wq_ref[...].astype(jnp.bfloat16)
        comm_wo[0] = wo_ref[...].astype(jnp.bfloat16)
        out_ref[...] = jnp.zeros((B, SQ, DM), jnp.float32)

        r = lax.broadcasted_iota(jnp.int32, (SQ, SKV), 0)
        c = lax.broadcasted_iota(jnp.int32, (SQ, SKV), 1)
        qb = my * (SQ // 64) + r // 64
        kb_blk = c // 64
        mask = (qb == kb_blk) | (kb_blk == 0) | (jnp.mod(qb + kb_blk, 3) == 0)

        x2d = x_ref[...].reshape(B * SQ, DM).astype(jnp.bfloat16)

        def kv_fetch(h):
            blk = jnp.mod(my - h, N_DEV)
            slot = jnp.mod(h, 2)
            kcp = pltpu.make_async_copy(
                k_hbm.at[:, :, pl.ds(blk * HQ_LOC, HQ_LOC), :],
                kbuf.at[slot], ksem.at[slot])
            vcp = pltpu.make_async_copy(
                v_hbm.at[:, :, pl.ds(blk * HQ_LOC, HQ_LOC), :],
                vbuf.at[slot], vsem.at[slot])
            kcp.start()
            vcp.start()

        def kv_wait(h):
            slot = jnp.mod(h, 2)
            pltpu.make_async_copy(k_hbm.at[:, :, pl.ds(0, HQ_LOC), :],
                                  kbuf.at[slot], ksem.at[slot]).wait()
            pltpu.make_async_copy(v_hbm.at[:, :, pl.ds(0, HQ_LOC), :],
                                  vbuf.at[slot], vsem.at[slot]).wait()

        kv_fetch(0)

        def step(h, _):
            send_slot = jnp.mod(h, 2)
            recv_slot = jnp.mod(h + 1, 2)

            rdma_wq = pltpu.make_async_remote_copy(
                src_ref=comm_wq.at[send_slot],
                dst_ref=comm_wq.at[recv_slot],
                send_sem=wq_send_sem.at[send_slot],
                recv_sem=wq_recv_sem.at[recv_slot],
                device_id=(right,),
                device_id_type=pl.DeviceIdType.MESH,
            )
            rdma_wo = pltpu.make_async_remote_copy(
                src_ref=comm_wo.at[send_slot],
                dst_ref=comm_wo.at[recv_slot],
                send_sem=wo_send_sem.at[send_slot],
                recv_sem=wo_recv_sem.at[recv_slot],
                device_id=(right,),
                device_id_type=pl.DeviceIdType.MESH,
            )

            @pl.when(h < N_DEV - 1)
            def _():
                rdma_wq.start()
                rdma_wo.start()
                kv_fetch(h + 1)

            kv_wait(h)

            wq_k = comm_wq[send_slot]
            wo_k = comm_wo[send_slot]
            q = jnp.dot(x2d, wq_k, preferred_element_type=jnp.float32)
            q4 = q.reshape(B, SQ, HQ_LOC, DH).astype(jnp.bfloat16)
            kv_k = kbuf[send_slot].astype(jnp.bfloat16)
            vv_k = vbuf[send_slot].astype(jnp.bfloat16)

            ctx_bs = []
            for b in range(B):
                ctx_h = []
                for hh in range(HQ_LOC):
                    q_bh = q4[b, :, hh, :]
                    k_bh = kv_k[b, :, hh, :]
                    s = lax.dot_general(
                        q_bh, k_bh, (((1,), (1,)), ((), ())),
                        preferred_element_type=jnp.float32) * 0.125
                    s = jnp.where(mask, s, -1e9)
                    mx = jnp.max(s, axis=1, keepdims=True)
                    w = jnp.exp(s - mx)
                    w = (w / jnp.sum(w, axis=1, keepdims=True)).astype(
                        jnp.bfloat16)
                    ctx_h.append(jnp.dot(w, vv_k[b, :, hh, :],
                                         preferred_element_type=jnp.float32))
                ctx_bs.append(jnp.concatenate(ctx_h, axis=1))
            ctx2d = jnp.stack(ctx_bs, axis=0).reshape(B * SQ, HD).astype(
                jnp.bfloat16)
            partial = jnp.dot(ctx2d, wo_k, preferred_element_type=jnp.float32)
            out_ref[...] = out_ref[...] + partial.reshape(B, SQ, DM)

            @pl.when(h < N_DEV - 1)
            def _():
                rdma_wq.wait()
                rdma_wo.wait()

            return 0

        lax.fori_loop(0, N_DEV, step, 0)

    return pl.pallas_call(
        body,
        out_shape=jax.ShapeDtypeStruct((B, SQ, DM), jnp.float32),
        in_specs=[
            pl.BlockSpec(memory_space=pltpu.VMEM),
            pl.BlockSpec(memory_space=pltpu.VMEM),
            pl.BlockSpec(memory_space=pltpu.MemorySpace.HBM),
            pl.BlockSpec(memory_space=pltpu.MemorySpace.HBM),
            pl.BlockSpec(memory_space=pltpu.VMEM),
        ],
        out_specs=pl.BlockSpec(memory_space=pltpu.VMEM),
        scratch_shapes=[
            pltpu.VMEM((2, DM, HD), jnp.bfloat16),
            pltpu.VMEM((2, HD, DM), jnp.bfloat16),
            pltpu.VMEM((2, B, SKV, HQ_LOC, DH), jnp.float32),
            pltpu.VMEM((2, B, SKV, HQ_LOC, DH), jnp.float32),
            pltpu.SemaphoreType.DMA((2,)),
            pltpu.SemaphoreType.DMA((2,)),
            pltpu.SemaphoreType.DMA((2,)),
            pltpu.SemaphoreType.DMA((2,)),
            pltpu.SemaphoreType.DMA((2,)),
            pltpu.SemaphoreType.DMA((2,)),
        ],
        compiler_params=pltpu.CompilerParams(collective_id=0),
    )(x, Wq, K_ext, V_ext, Wo)


# device time: 260353 ns/iter; 1.7647x vs baseline; 1.0894x over previous
import jax
import jax.numpy as jnp
from jax import lax
from jax.experimental import pallas as pl
from jax.experimental.pallas import tpu as pltpu

N_DEV = 32
N_HOP = 16
B = 2
SQ = 256
SKV = 256
HQ_LOC = 4
DH = 64
DM = 512
HD = HQ_LOC * DH


def kernel(x, Wq, K_ext, V_ext, Wo):
    def body(x_ref, wq_ref, k_hbm, v_hbm, wo_ref, out_ref,
             comm_wq_r, comm_wo_r, comm_wq_l, comm_wo_l, kbuf, vbuf,
             sems_r, sems_l, ksem, vsem):
        my = lax.axis_index("i")
        left = jnp.mod(my - 1, N_DEV)
        right = jnp.mod(my + 1, N_DEV)

        bsem = pltpu.get_barrier_semaphore()
        pl.semaphore_signal(bsem, inc=1, device_id=(left,),
                            device_id_type=pl.DeviceIdType.MESH)
        pl.semaphore_signal(bsem, inc=1, device_id=(right,),
                            device_id_type=pl.DeviceIdType.MESH)
        pl.semaphore_wait(bsem, 2)

        comm_wq_r[0] = wq_ref[...].astype(jnp.bfloat16)
        comm_wo_r[0] = wo_ref[...].astype(jnp.bfloat16)
        comm_wq_l[0] = wq_ref[...].astype(jnp.bfloat16)
        comm_wo_l[0] = wo_ref[...].astype(jnp.bfloat16)
        out_ref[...] = jnp.zeros((B, SQ, DM), jnp.float32)

        r = lax.broadcasted_iota(jnp.int32, (SQ, SKV), 0)
        c = lax.broadcasted_iota(jnp.int32, (SQ, SKV), 1)
        qb = my * (SQ // 64) + r // 64
        kb_blk = c // 64
        mask = (qb == kb_blk) | (kb_blk == 0) | (jnp.mod(qb + kb_blk, 3) == 0)

        x2d = x_ref[...].reshape(B * SQ, DM).astype(jnp.bfloat16)

        def kv_fetch(blk, slot, d):
            kcp = pltpu.make_async_copy(
                k_hbm.at[:, :, pl.ds(blk * HQ_LOC, HQ_LOC), :],
                kbuf.at[slot, d], ksem.at[slot, d])
            vcp = pltpu.make_async_copy(
                v_hbm.at[:, :, pl.ds(blk * HQ_LOC, HQ_LOC), :],
                vbuf.at[slot, d], vsem.at[slot, d])
            kcp.start()
            vcp.start()

        def kv_wait(slot, d):
            pltpu.make_async_copy(k_hbm.at[:, :, pl.ds(0, HQ_LOC), :],
                                  kbuf.at[slot, d], ksem.at[slot, d]).wait()
            pltpu.make_async_copy(v_hbm.at[:, :, pl.ds(0, HQ_LOC), :],
                                  vbuf.at[slot, d], vsem.at[slot, d]).wait()

        def compute_block(wq_k, wo_k, kv_k, vv_k):
            q = jnp.dot(x2d, wq_k, preferred_element_type=jnp.float32)
            q4 = q.reshape(B, SQ, HQ_LOC, DH).astype(jnp.bfloat16)
            kv_b = kv_k.astype(jnp.bfloat16)
            vv_b = vv_k.astype(jnp.bfloat16)
            ctx_bs = []
            for b in range(B):
                ctx_h = []
                for hh in range(HQ_LOC):
                    q_bh = q4[b, :, hh, :]
                    k_bh = kv_b[b, :, hh, :]
                    s = lax.dot_general(
                        q_bh, k_bh, (((1,), (1,)), ((), ())),
                        preferred_element_type=jnp.float32) * 0.125
                    s = jnp.where(mask, s, -1e9)
                    mx = jnp.max(s, axis=1, keepdims=True)
                    w = jnp.exp(s - mx)
                    w = (w / jnp.sum(w, axis=1, keepdims=True)).astype(
                        jnp.bfloat16)
                    ctx_h.append(jnp.dot(w, vv_b[b, :, hh, :],
                                         preferred_element_type=jnp.float32))
                ctx_bs.append(jnp.concatenate(ctx_h, axis=1))
            ctx2d = jnp.stack(ctx_bs, axis=0).reshape(B * SQ, HD).astype(
                jnp.bfloat16)
            partial = jnp.dot(ctx2d, wo_k, preferred_element_type=jnp.float32)
            out_ref[...] = out_ref[...] + partial.reshape(B, SQ, DM)

        def mk_rdma(wq_buf, wo_buf, sems, send_slot, recv_slot, dst):
            rdma_wq = pltpu.make_async_remote_copy(
                src_ref=wq_buf.at[send_slot],
                dst_ref=wq_buf.at[recv_slot],
                send_sem=sems.at[0, send_slot],
                recv_sem=sems.at[1, recv_slot],
                device_id=(dst,),
                device_id_type=pl.DeviceIdType.MESH,
            )
            rdma_wo = pltpu.make_async_remote_copy(
                src_ref=wo_buf.at[send_slot],
                dst_ref=wo_buf.at[recv_slot],
                send_sem=sems.at[2, send_slot],
                recv_sem=sems.at[3, recv_slot],
                device_id=(dst,),
                device_id_type=pl.DeviceIdType.MESH,
            )
            return rdma_wq, rdma_wo

        kv_fetch(my, 0, 0)

        def step(h, _):
            send_slot = jnp.mod(h, 2)
            recv_slot = jnp.mod(h + 1, 2)

            wq_r, wo_r = mk_rdma(comm_wq_r, comm_wo_r, sems_r,
                                 send_slot, recv_slot, right)
            wq_l, wo_l = mk_rdma(comm_wq_l, comm_wo_l, sems_l,
                                 send_slot, recv_slot, left)

            wq_r.start()
            wo_r.start()
            @pl.when(h < N_HOP - 1)
            def _():
                wq_l.start()
                wo_l.start()

            @pl.when(h < N_HOP - 1)
            def _():
                kv_fetch(jnp.mod(my - h - 1, N_DEV), recv_slot, 0)
                kv_fetch(jnp.mod(my + h + 1, N_DEV), recv_slot, 1)

            @pl.when(h == N_HOP - 1)
            def _():
                kv_fetch(jnp.mod(my - N_HOP, N_DEV), recv_slot, 0)

            kv_wait(send_slot, 0)
            compute_block(comm_wq_r[send_slot], comm_wo_r[send_slot],
                          kbuf[send_slot, 0], vbuf[send_slot, 0])

            @pl.when(h > 0)
            def _():
                kv_wait(send_slot, 1)
                compute_block(comm_wq_l[send_slot], comm_wo_l[send_slot],
                              kbuf[send_slot, 1], vbuf[send_slot, 1])

            wq_r.wait()
            wo_r.wait()

            @pl.when(h < N_HOP - 1)
            def _():
                wq_l.wait()
                wo_l.wait()

            return 0

        lax.fori_loop(0, N_HOP, step, 0)

        kv_wait(0, 0)
        compute_block(comm_wq_r[0], comm_wo_r[0], kbuf[0, 0], vbuf[0, 0])

    return pl.pallas_call(
        body,
        out_shape=jax.ShapeDtypeStruct((B, SQ, DM), jnp.float32),
        in_specs=[
            pl.BlockSpec(memory_space=pltpu.VMEM),
            pl.BlockSpec(memory_space=pltpu.VMEM),
            pl.BlockSpec(memory_space=pltpu.MemorySpace.HBM),
            pl.BlockSpec(memory_space=pltpu.MemorySpace.HBM),
            pl.BlockSpec(memory_space=pltpu.VMEM),
        ],
        out_specs=pl.BlockSpec(memory_space=pltpu.VMEM),
        scratch_shapes=[
            pltpu.VMEM((2, DM, HD), jnp.bfloat16),
            pltpu.VMEM((2, HD, DM), jnp.bfloat16),
            pltpu.VMEM((2, DM, HD), jnp.bfloat16),
            pltpu.VMEM((2, HD, DM), jnp.bfloat16),
            pltpu.VMEM((2, 2, B, SKV, HQ_LOC, DH), jnp.float32),
            pltpu.VMEM((2, 2, B, SKV, HQ_LOC, DH), jnp.float32),
            pltpu.SemaphoreType.DMA((4, 2)),
            pltpu.SemaphoreType.DMA((4, 2)),
            pltpu.SemaphoreType.DMA((2, 2)),
            pltpu.SemaphoreType.DMA((2, 2)),
        ],
        compiler_params=pltpu.CompilerParams(collective_id=0),
    )(x, Wq, K_ext, V_ext, Wo)


# device time: 255934 ns/iter; 1.7952x vs baseline; 1.0173x over previous
import jax
import jax.numpy as jnp
from jax import lax
from jax.experimental import pallas as pl
from jax.experimental.pallas import tpu as pltpu

N_DEV = 32
N_HOP = 16
B = 2
SQ = 256
SKV = 256
HQ_LOC = 4
DH = 64
DM = 512
HD = HQ_LOC * DH


def kernel(x, Wq, K_ext, V_ext, Wo):
    def body(x_ref, wq_ref, k_hbm, v_hbm, wo_ref, out_ref,
             comm_wq_r, comm_wo_r, comm_wq_l, comm_wo_l, kbuf, vbuf,
             sems_r, sems_l, ksem, vsem):
        my = lax.axis_index("i")
        left = jnp.mod(my - 1, N_DEV)
        right = jnp.mod(my + 1, N_DEV)

        bsem = pltpu.get_barrier_semaphore()
        pl.semaphore_signal(bsem, inc=1, device_id=(left,),
                            device_id_type=pl.DeviceIdType.MESH)
        pl.semaphore_signal(bsem, inc=1, device_id=(right,),
                            device_id_type=pl.DeviceIdType.MESH)
        pl.semaphore_wait(bsem, 2)

        comm_wq_r[0] = wq_ref[...].astype(jnp.bfloat16)
        comm_wo_r[0] = wo_ref[...].astype(jnp.bfloat16)
        comm_wq_l[0] = wq_ref[...].astype(jnp.bfloat16)
        comm_wo_l[0] = wo_ref[...].astype(jnp.bfloat16)
        out_ref[...] = jnp.zeros((B, SQ, DM), jnp.float32)

        r = lax.broadcasted_iota(jnp.int32, (SQ, SKV), 0)
        c = lax.broadcasted_iota(jnp.int32, (SQ, SKV), 1)
        qb = my * (SQ // 64) + r // 64
        kb_blk = c // 64
        mask = (qb == kb_blk) | (kb_blk == 0) | (jnp.mod(qb + kb_blk, 3) == 0)
        bias = jnp.where(mask, 0.0, -1e9).astype(jnp.float32)

        x2d = (x_ref[...].reshape(B * SQ, DM) * 0.125).astype(jnp.bfloat16)

        def kv_fetch(blk, slot, d):
            kcp = pltpu.make_async_copy(
                k_hbm.at[:, :, pl.ds(blk * HQ_LOC, HQ_LOC), :],
                kbuf.at[slot, d], ksem.at[slot, d])
            vcp = pltpu.make_async_copy(
                v_hbm.at[:, :, pl.ds(blk * HQ_LOC, HQ_LOC), :],
                vbuf.at[slot, d], vsem.at[slot, d])
            kcp.start()
            vcp.start()

        def kv_wait(slot, d):
            pltpu.make_async_copy(k_hbm.at[:, :, pl.ds(0, HQ_LOC), :],
                                  kbuf.at[slot, d], ksem.at[slot, d]).wait()
            pltpu.make_async_copy(v_hbm.at[:, :, pl.ds(0, HQ_LOC), :],
                                  vbuf.at[slot, d], vsem.at[slot, d]).wait()

        def compute_block(wq_k, wo_k, kv_k, vv_k):
            q = jnp.dot(x2d, wq_k, preferred_element_type=jnp.float32)
            q4 = q.reshape(B, SQ, HQ_LOC, DH).astype(jnp.bfloat16)
            kv_b = kv_k.astype(jnp.bfloat16)
            vv_b = vv_k.astype(jnp.bfloat16)
            ctx_bs = []
            for b in range(B):
                ctx_h = []
                for hh in range(HQ_LOC):
                    q_bh = q4[b, :, hh, :]
                    k_bh = kv_b[b, :, hh, :]
                    s = lax.dot_general(
                        q_bh, k_bh, (((1,), (1,)), ((), ())),
                        preferred_element_type=jnp.float32)
                    w = jnp.exp(s + bias)
                    rinv = 1.0 / jnp.sum(w, axis=1, keepdims=True)
                    ctx_bh = jnp.dot(w.astype(jnp.bfloat16),
                                     vv_b[b, :, hh, :],
                                     preferred_element_type=jnp.float32)
                    ctx_h.append(ctx_bh * rinv)
                ctx_bs.append(jnp.concatenate(ctx_h, axis=1))
            ctx2d = jnp.stack(ctx_bs, axis=0).reshape(B * SQ, HD).astype(
                jnp.bfloat16)
            partial = jnp.dot(ctx2d, wo_k, preferred_element_type=jnp.float32)
            out_ref[...] = out_ref[...] + partial.reshape(B, SQ, DM)

        def mk_rdma(wq_buf, wo_buf, sems, send_slot, recv_slot, dst):
            rdma_wq = pltpu.make_async_remote_copy(
                src_ref=wq_buf.at[send_slot],
                dst_ref=wq_buf.at[recv_slot],
                send_sem=sems.at[0, send_slot],
                recv_sem=sems.at[1, recv_slot],
                device_id=(dst,),
                device_id_type=pl.DeviceIdType.MESH,
            )
            rdma_wo = pltpu.make_async_remote_copy(
                src_ref=wo_buf.at[send_slot],
                dst_ref=wo_buf.at[recv_slot],
                send_sem=sems.at[2, send_slot],
                recv_sem=sems.at[3, recv_slot],
                device_id=(dst,),
                device_id_type=pl.DeviceIdType.MESH,
            )
            return rdma_wq, rdma_wo

        kv_fetch(my, 0, 0)

        def step(h, _):
            send_slot = jnp.mod(h, 2)
            recv_slot = jnp.mod(h + 1, 2)

            wq_r, wo_r = mk_rdma(comm_wq_r, comm_wo_r, sems_r,
                                 send_slot, recv_slot, right)
            wq_l, wo_l = mk_rdma(comm_wq_l, comm_wo_l, sems_l,
                                 send_slot, recv_slot, left)

            wq_r.start()
            wo_r.start()
            @pl.when(h < N_HOP - 1)
            def _():
                wq_l.start()
                wo_l.start()

            @pl.when(h < N_HOP - 1)
            def _():
                kv_fetch(jnp.mod(my - h - 1, N_DEV), recv_slot, 0)
                kv_fetch(jnp.mod(my + h + 1, N_DEV), recv_slot, 1)

            @pl.when(h == N_HOP - 1)
            def _():
                kv_fetch(jnp.mod(my - N_HOP, N_DEV), recv_slot, 0)

            kv_wait(send_slot, 0)
            compute_block(comm_wq_r[send_slot], comm_wo_r[send_slot],
                          kbuf[send_slot, 0], vbuf[send_slot, 0])

            @pl.when(h > 0)
            def _():
                kv_wait(send_slot, 1)
                compute_block(comm_wq_l[send_slot], comm_wo_l[send_slot],
                              kbuf[send_slot, 1], vbuf[send_slot, 1])

            wq_r.wait()
            wo_r.wait()

            @pl.when(h < N_HOP - 1)
            def _():
                wq_l.wait()
                wo_l.wait()

            return 0

        lax.fori_loop(0, N_HOP, step, 0)

        kv_wait(0, 0)
        compute_block(comm_wq_r[0], comm_wo_r[0], kbuf[0, 0], vbuf[0, 0])

    return pl.pallas_call(
        body,
        out_shape=jax.ShapeDtypeStruct((B, SQ, DM), jnp.float32),
        in_specs=[
            pl.BlockSpec(memory_space=pltpu.VMEM),
            pl.BlockSpec(memory_space=pltpu.VMEM),
            pl.BlockSpec(memory_space=pltpu.MemorySpace.HBM),
            pl.BlockSpec(memory_space=pltpu.MemorySpace.HBM),
            pl.BlockSpec(memory_space=pltpu.VMEM),
        ],
        out_specs=pl.BlockSpec(memory_space=pltpu.VMEM),
        scratch_shapes=[
            pltpu.VMEM((2, DM, HD), jnp.bfloat16),
            pltpu.VMEM((2, HD, DM), jnp.bfloat16),
            pltpu.VMEM((2, DM, HD), jnp.bfloat16),
            pltpu.VMEM((2, HD, DM), jnp.bfloat16),
            pltpu.VMEM((2, 2, B, SKV, HQ_LOC, DH), jnp.float32),
            pltpu.VMEM((2, 2, B, SKV, HQ_LOC, DH), jnp.float32),
            pltpu.SemaphoreType.DMA((4, 2)),
            pltpu.SemaphoreType.DMA((4, 2)),
            pltpu.SemaphoreType.DMA((2, 2)),
            pltpu.SemaphoreType.DMA((2, 2)),
        ],
        compiler_params=pltpu.CompilerParams(collective_id=0),
    )(x, Wq, K_ext, V_ext, Wo)
